# baseline (speedup 1.0000x reference)
import functools

import jax
import jax.numpy as jnp
from jax import lax
from jax.experimental import pallas as pl
from jax.experimental.pallas import tpu as pltpu
from jax.experimental.pallas import tpu_sc as plsc

_NC, _NS, _L = 2, 16, 16
_NW = _NC * _NS
_C = 16


def _make_sc_kernel(bs, sl, d):
    spw = sl // _NW
    nchunk = spw // _C
    mesh = plsc.VectorSubcoreMesh(core_axis_name="c", subcore_axis_name="s")

    @functools.partial(
        pl.kernel,
        out_type=jax.ShapeDtypeStruct((bs * sl, d), jnp.float32),
        mesh=mesh,
        scratch_types=[
            pltpu.VMEM((_C, d), jnp.float32),
            pltpu.VMEM((_C, d), jnp.float32),
            pltpu.VMEM((bs * spw,), jnp.float32),
        ],
    )
    def sc_pe(x_hbm, mask_hbm, pe_hbm, out_hbm, pe_v, x_v, m_v):
        wid = lax.axis_index("s") * _NC + lax.axis_index("c")
        s0 = wid * spw
        for b in range(bs):
            pltpu.sync_copy(
                mask_hbm.at[pl.ds(b * sl + s0, spw)],
                m_v.at[pl.ds(b * spw, spw)],
            )

        def chunk_body(ci, carry):
            base_s = s0 + ci * _C
            pltpu.sync_copy(pe_hbm.at[pl.ds(base_s, _C)], pe_v)
            for b in range(bs):
                pltpu.sync_copy(x_hbm.at[pl.ds(b * sl + base_s, _C)], x_v)

                m_vec = m_v[pl.ds(b * spw + ci * _C, _L)]

                def row_body(j, carry2):
                    midx = jnp.broadcast_to(j, (_L,)).astype(jnp.int32)
                    m16 = m_vec.at[midx].get(mode="promise_in_bounds")

                    def col_body(k, carry3):
                        s16 = pl.ds(k * _L, _L)
                        x_v[j, s16] = (x_v[j, s16] + pe_v[j, s16]) * m16
                        return carry3

                    return lax.fori_loop(0, d // _L, col_body, carry2)

                lax.fori_loop(0, _C, row_body, 0)
                pltpu.sync_copy(x_v, out_hbm.at[pl.ds(b * sl + base_s, _C)])
            return carry

        lax.fori_loop(0, nchunk, chunk_body, 0)

    return sc_pe


def kernel(x, mask, pos_emb):
    bs, sl, d = x.shape
    out = _make_sc_kernel(bs, sl, d)(
        x.reshape(bs * sl, d), mask.reshape(bs * sl), pos_emb
    )
    return out.reshape(bs, sl, d)


# speedup vs baseline: 5.4920x; 5.4920x over previous
import jax
import jax.numpy as jnp
from jax.experimental import pallas as pl

S_BLK = 1024


def _pe_kernel(x_ref, mask_ref, pe_ref, out_ref):
    m = mask_ref[0, 0, 0, :]
    out_ref[...] = (x_ref[...] + pe_ref[...]) * m[:, None]


def kernel(x, mask, pos_emb):
    bs, sl, d = x.shape
    grid = (sl // S_BLK, bs)
    mask4 = mask.reshape(bs, sl // S_BLK, 1, S_BLK)
    return pl.pallas_call(
        _pe_kernel,
        grid=grid,
        in_specs=[
            pl.BlockSpec((1, S_BLK, d), lambda s, b: (b, s, 0)),
            pl.BlockSpec((1, 1, 1, S_BLK), lambda s, b: (b, s, 0, 0)),
            pl.BlockSpec((S_BLK, d), lambda s, b: (s, 0)),
        ],
        out_specs=pl.BlockSpec((1, S_BLK, d), lambda s, b: (b, s, 0)),
        out_shape=jax.ShapeDtypeStruct((bs, sl, d), x.dtype),
    )(x, mask4, pos_emb)
